# baseline (speedup 1.0000x reference)
import functools

import jax
import jax.numpy as jnp
from jax import lax
from jax.experimental import pallas as pl
from jax.experimental.pallas import tpu as pltpu
from jax.experimental.pallas import tpu_sc as plsc

NUM_LEVELS = 4
TABLE_W = 1001
BATCH = 4096
SEQ_L = 200

NUM_WORKERS = 32
COLS_PER_W = BATCH // NUM_WORKERS
VREGS_PER_ROW = COLS_PER_W // 16
_NCHUNK = 5
_CHUNK = SEQ_L // _NCHUNK

_MESH = plsc.VectorSubcoreMesh(core_axis_name="c", subcore_axis_name="s")


@functools.partial(
    pl.kernel,
    out_type=jax.ShapeDtypeStruct((SEQ_L, BATCH), jnp.float32),
    mesh=_MESH,
    scratch_types=[
        pltpu.VMEM((NUM_LEVELS, TABLE_W), jnp.float32),
        pltpu.VMEM((COLS_PER_W,), jnp.int32),
        pltpu.VMEM((_CHUNK, COLS_PER_W), jnp.int32),
        pltpu.VMEM((_CHUNK, COLS_PER_W), jnp.int32),
        pltpu.VMEM((COLS_PER_W * NUM_LEVELS,), jnp.float32),
        pltpu.VMEM((_CHUNK, COLS_PER_W), jnp.float32),
        pltpu.VMEM((_CHUNK, COLS_PER_W), jnp.float32),
        pltpu.SemaphoreType.DMA,
        pltpu.SemaphoreType.DMA,
        pltpu.SemaphoreType.DMA,
        pltpu.SemaphoreType.DMA,
    ],
    compiler_params=pltpu.CompilerParams(needs_layout_passes=False),
)
def _sc_gather(t_hbm, labels_hbm, table_hbm, out_hbm,
               table_v, t_v, lbl0_v, lbl1_v, rates_v, out0_v, out1_v,
               lsem0, lsem1, osem0, osem1):
    wid = lax.axis_index("s") * 2 + lax.axis_index("c")
    col0 = wid * COLS_PER_W
    lbufs, obufs = (lbl0_v, lbl1_v), (out0_v, out1_v)
    lsems, osems = (lsem0, lsem1), (osem0, osem1)

    def fetch(g):
        return pltpu.async_copy(
            labels_hbm.at[pl.ds(g * _CHUNK, _CHUNK), pl.ds(col0, COLS_PER_W)],
            lbufs[g % 2], lsems[g % 2])

    in_flight = [fetch(0), fetch(1)]

    pltpu.sync_copy(table_hbm, table_v)
    pltpu.sync_copy(t_hbm.at[pl.ds(col0, COLS_PER_W)], t_v)

    lane = lax.iota(jnp.int32, 16)

    @plsc.parallel_loop(0, COLS_PER_W * NUM_LEVELS, 16, unroll=2)
    def rates_stage(p0):
        base = pl.multiple_of(p0, 16)
        p = p0 + lane
        b = lax.shift_right_logical(p, 2)
        k = lax.bitwise_and(p, 3)
        tb = plsc.load_gather(t_v, [b])
        rates_v[pl.ds(base, 16)] = plsc.load_gather(table_v, [k, tb])

    bases = [lax.shift_left(j * 16 + lane, 2) for j in range(VREGS_PER_ROW)]

    out_flight = [None, None]
    for g in range(_NCHUNK):
        in_flight[g % 2].wait()
        if g + 2 < _NCHUNK:
            in_flight[g % 2] = fetch(g + 2)
        if out_flight[g % 2] is not None:
            out_flight[g % 2].wait()
        lbl_v, out_v = lbufs[g % 2], obufs[g % 2]

        @plsc.parallel_loop(0, _CHUNK, 1, unroll=2)
        def apply_stage(l):
            for j in range(VREGS_PER_ROW):
                lbl = lbl_v[l, pl.ds(j * 16, 16)]
                out_v[l, pl.ds(j * 16, 16)] = plsc.load_gather(
                    rates_v, [bases[j] + lbl])

        out_flight[g % 2] = pltpu.async_copy(
            out_v,
            out_hbm.at[pl.ds(g * _CHUNK, _CHUNK), pl.ds(col0, COLS_PER_W)],
            osems[g % 2])

    for h in out_flight:
        if h is not None:
            h.wait()


def kernel(t, hierarchy_labels, masking_rates):
    out_t = _sc_gather(t.astype(jnp.int32),
                       hierarchy_labels.astype(jnp.int32).T,
                       masking_rates.astype(jnp.float32))
    return out_t.T


# speedup vs baseline: 1.0078x; 1.0078x over previous
import functools

import jax
import jax.numpy as jnp
from jax import lax
from jax.experimental import pallas as pl
from jax.experimental.pallas import tpu as pltpu
from jax.experimental.pallas import tpu_sc as plsc

NUM_LEVELS = 4
TABLE_W = 1001
BATCH = 4096
SEQ_L = 200

NUM_WORKERS = 16
COLS_PER_W = BATCH // NUM_WORKERS
VREGS_PER_ROW = COLS_PER_W // 16
_NCHUNK = 5
_CHUNK = SEQ_L // _NCHUNK

_MESH = plsc.VectorSubcoreMesh(core_axis_name="c", subcore_axis_name="s", num_cores=1)


@functools.partial(
    pl.kernel,
    out_type=jax.ShapeDtypeStruct((SEQ_L, BATCH), jnp.float32),
    mesh=_MESH,
    scratch_types=[
        pltpu.VMEM((NUM_LEVELS, TABLE_W), jnp.float32),
        pltpu.VMEM((COLS_PER_W,), jnp.int32),
        pltpu.VMEM((_CHUNK, COLS_PER_W), jnp.int32),
        pltpu.VMEM((_CHUNK, COLS_PER_W), jnp.int32),
        pltpu.VMEM((COLS_PER_W * NUM_LEVELS,), jnp.float32),
        pltpu.VMEM((_CHUNK, COLS_PER_W), jnp.float32),
        pltpu.VMEM((_CHUNK, COLS_PER_W), jnp.float32),
        pltpu.SemaphoreType.DMA,
        pltpu.SemaphoreType.DMA,
        pltpu.SemaphoreType.DMA,
        pltpu.SemaphoreType.DMA,
    ],
    compiler_params=pltpu.CompilerParams(needs_layout_passes=False),
)
def _sc_gather(t_hbm, labels_hbm, table_hbm, out_hbm,
               table_v, t_v, lbl0_v, lbl1_v, rates_v, out0_v, out1_v,
               lsem0, lsem1, osem0, osem1):
    wid = lax.axis_index("s")
    col0 = wid * COLS_PER_W
    lbufs, obufs = (lbl0_v, lbl1_v), (out0_v, out1_v)
    lsems, osems = (lsem0, lsem1), (osem0, osem1)

    def fetch(g):
        return pltpu.async_copy(
            labels_hbm.at[pl.ds(g * _CHUNK, _CHUNK), pl.ds(col0, COLS_PER_W)],
            lbufs[g % 2], lsems[g % 2])

    in_flight = [fetch(0), fetch(1)]

    pltpu.sync_copy(table_hbm, table_v)
    pltpu.sync_copy(t_hbm.at[pl.ds(col0, COLS_PER_W)], t_v)

    lane = lax.iota(jnp.int32, 16)

    @plsc.parallel_loop(0, COLS_PER_W * NUM_LEVELS, 16, unroll=2)
    def rates_stage(p0):
        base = pl.multiple_of(p0, 16)
        p = p0 + lane
        b = lax.shift_right_logical(p, 2)
        k = lax.bitwise_and(p, 3)
        tb = plsc.load_gather(t_v, [b])
        rates_v[pl.ds(base, 16)] = plsc.load_gather(table_v, [k, tb])

    bases = [lax.shift_left(j * 16 + lane, 2) for j in range(VREGS_PER_ROW)]

    out_flight = [None, None]
    for g in range(_NCHUNK):
        in_flight[g % 2].wait()
        if g + 2 < _NCHUNK:
            in_flight[g % 2] = fetch(g + 2)
        if out_flight[g % 2] is not None:
            out_flight[g % 2].wait()
        lbl_v, out_v = lbufs[g % 2], obufs[g % 2]

        @plsc.parallel_loop(0, _CHUNK, 1)
        def apply_stage(l):
            for j in range(VREGS_PER_ROW):
                lbl = lbl_v[l, pl.ds(j * 16, 16)]
                out_v[l, pl.ds(j * 16, 16)] = plsc.load_gather(
                    rates_v, [bases[j] + lbl])

        out_flight[g % 2] = pltpu.async_copy(
            out_v,
            out_hbm.at[pl.ds(g * _CHUNK, _CHUNK), pl.ds(col0, COLS_PER_W)],
            osems[g % 2])

    for h in out_flight:
        if h is not None:
            h.wait()


def kernel(t, hierarchy_labels, masking_rates):
    out_t = _sc_gather(t.astype(jnp.int32),
                       hierarchy_labels.astype(jnp.int32).T,
                       masking_rates.astype(jnp.float32))
    return out_t.T


# speedup vs baseline: 1.0214x; 1.0135x over previous
import functools

import jax
import jax.numpy as jnp
from jax import lax
from jax.experimental import pallas as pl
from jax.experimental.pallas import tpu as pltpu
from jax.experimental.pallas import tpu_sc as plsc

NUM_LEVELS = 4
TABLE_W = 1001
BATCH = 4096
SEQ_L = 200

NUM_WORKERS = 32
COLS_PER_W = BATCH // NUM_WORKERS
VREGS_PER_ROW = COLS_PER_W // 16
_NCHUNK = 5
_CHUNK = SEQ_L // _NCHUNK

_MESH = plsc.VectorSubcoreMesh(core_axis_name="c", subcore_axis_name="s")


@functools.partial(
    pl.kernel,
    out_type=jax.ShapeDtypeStruct((SEQ_L, BATCH), jnp.float32),
    mesh=_MESH,
    scratch_types=[
        pltpu.VMEM((NUM_LEVELS, TABLE_W), jnp.float32),
        pltpu.VMEM((COLS_PER_W,), jnp.int32),
        pltpu.VMEM((_CHUNK, COLS_PER_W), jnp.int32),
        pltpu.VMEM((_CHUNK, COLS_PER_W), jnp.int32),
        pltpu.VMEM((COLS_PER_W * NUM_LEVELS,), jnp.float32),
        pltpu.VMEM((_CHUNK, COLS_PER_W), jnp.float32),
        pltpu.VMEM((_CHUNK, COLS_PER_W), jnp.float32),
        pltpu.SemaphoreType.DMA,
        pltpu.SemaphoreType.DMA,
        pltpu.SemaphoreType.DMA,
        pltpu.SemaphoreType.DMA,
    ],
    compiler_params=pltpu.CompilerParams(needs_layout_passes=False),
)
def _sc_gather(t_hbm, labels_hbm, table_hbm, out_hbm,
               table_v, t_v, lbl0_v, lbl1_v, rates_v, out0_v, out1_v,
               lsem0, lsem1, osem0, osem1):
    wid = lax.axis_index("s") * 2 + lax.axis_index("c")
    col0 = wid * COLS_PER_W
    lbufs, obufs = (lbl0_v, lbl1_v), (out0_v, out1_v)
    lsems, osems = (lsem0, lsem1), (osem0, osem1)

    def fetch(g):
        return pltpu.async_copy(
            labels_hbm.at[pl.ds(g * _CHUNK, _CHUNK), pl.ds(col0, COLS_PER_W)],
            lbufs[g % 2], lsems[g % 2])

    in_flight = [fetch(0), fetch(1)]

    pltpu.sync_copy(table_hbm, table_v)
    pltpu.sync_copy(t_hbm.at[pl.ds(col0, COLS_PER_W)], t_v)

    lane = lax.iota(jnp.int32, 16)

    @plsc.parallel_loop(0, COLS_PER_W * NUM_LEVELS, 16, unroll=2)
    def rates_stage(p0):
        base = pl.multiple_of(p0, 16)
        p = p0 + lane
        b = lax.shift_right_logical(p, 2)
        k = lax.bitwise_and(p, 3)
        tb = plsc.load_gather(t_v, [b])
        rates_v[pl.ds(base, 16)] = plsc.load_gather(table_v, [k, tb])

    bases = [lax.shift_left(j * 16 + lane, 2) for j in range(VREGS_PER_ROW)]

    out_flight = [None, None]
    for g in range(_NCHUNK):
        in_flight[g % 2].wait()
        if g + 2 < _NCHUNK:
            in_flight[g % 2] = fetch(g + 2)
        if out_flight[g % 2] is not None:
            out_flight[g % 2].wait()
        lbl_v, out_v = lbufs[g % 2], obufs[g % 2]

        @plsc.parallel_loop(0, _CHUNK, 1)
        def apply_stage(l):
            for j in range(VREGS_PER_ROW):
                lbl = lbl_v[l, pl.ds(j * 16, 16)]
                out_v[l, pl.ds(j * 16, 16)] = plsc.load_gather(
                    rates_v, [bases[j] + lbl])

        out_flight[g % 2] = pltpu.async_copy(
            out_v,
            out_hbm.at[pl.ds(g * _CHUNK, _CHUNK), pl.ds(col0, COLS_PER_W)],
            osems[g % 2])

    for h in out_flight:
        if h is not None:
            h.wait()


def kernel(t, hierarchy_labels, masking_rates):
    out_t = _sc_gather(t.astype(jnp.int32),
                       hierarchy_labels.astype(jnp.int32).T,
                       masking_rates.astype(jnp.float32))
    return out_t.T
